# baseline (speedup 1.0000x reference)
import jax
import jax.numpy as jnp
from jax.experimental import pallas as pl
from jax.experimental.pallas import tpu as pltpu

ROWS = 32
H = 512
W_DIM = 512
C = 32
NBLK = H // ROWS
MARGIN = 128
NLANES = ROWS * W_DIM
NDATA = (ROWS + 2) * W_DIM
XW = MARGIN + NDATA + 2 * MARGIN


def _row(r):
    return MARGIN + r * W_DIM


def _conv_body(x_ref, w2_ref, b_ref, mm_ref, mp_ref, out_ref, slabs_ref):
    i = pl.program_id(1)

    @pl.when(i == 0)
    def _():
        slabs_ref[...] = jnp.zeros_like(slabs_ref)

    par = jax.lax.rem(i, 2)
    slab_ref = slabs_ref.at[1 - par]
    stage_ref = slabs_ref.at[par]

    xt = jnp.swapaxes(x_ref[0], 0, 1)

    @pl.when(i > 0)
    def _():
        @pl.when(i < NBLK)
        def _():
            slab_ref[C:2 * C, _row(ROWS + 1):_row(ROWS + 2)] = xt[0]

        @pl.when(i == NBLK)
        def _():
            slab_ref[C:2 * C, _row(ROWS + 1):_row(ROWS + 2)] = jnp.zeros(
                (C, W_DIM), jnp.float32)

        b1 = slab_ref[C:2 * C, :]
        slab_ref[0:C, :] = pltpu.roll(b1, 1, 1) * mm_ref[...]
        slab_ref[2 * C:3 * C, :] = pltpu.roll(b1, XW - 1, 1) * mp_ref[...]

        p = jax.lax.dot_general(
            w2_ref[...], slab_ref[:, MARGIN:MARGIN + NDATA],
            (((1,), (0,)), ((), ())),
            preferred_element_type=jnp.float32)
        bias = b_ref[...]
        out = (p[0:C, 0:NLANES]
               + p[C:2 * C, W_DIM:NLANES + W_DIM]
               + p[2 * C:3 * C, 2 * W_DIM:NLANES + 2 * W_DIM])
        res3 = jnp.stack(
            [out[:, r * W_DIM:(r + 1) * W_DIM] + bias for r in range(ROWS)],
            axis=0)
        out_ref[0] = jnp.swapaxes(res3, 0, 1)

    @pl.when(i < NBLK)
    def _():
        @pl.when(i > 0)
        def _():
            stage_ref[C:2 * C, _row(0):_row(1)] = (
                slab_ref[C:2 * C, _row(ROWS):_row(ROWS + 1)])

        for r in range(ROWS):
            stage_ref[C:2 * C, _row(r + 1):_row(r + 2)] = xt[r]


def kernel(x, W, b):
    n = x.shape[0]
    w2 = jnp.transpose(W, (2, 0, 3, 1)).reshape(96, 96).astype(jnp.float32)
    bb = jnp.broadcast_to(b[:, None], (C, W_DIM))
    lane = jnp.arange(XW, dtype=jnp.int32)[None, :]
    mask_m = jnp.where((lane - MARGIN) % W_DIM == 0, 0.0, 1.0).astype(jnp.float32)
    mask_p = jnp.where((lane - MARGIN) % W_DIM == W_DIM - 1, 0.0, 1.0).astype(
        jnp.float32)

    return pl.pallas_call(
        _conv_body,
        grid=(n, NBLK + 1),
        in_specs=[
            pl.BlockSpec((1, C, ROWS, W_DIM),
                         lambda nn, ii: (nn, 0, jnp.minimum(ii, NBLK - 1), 0)),
            pl.BlockSpec((96, 96), lambda nn, ii: (0, 0)),
            pl.BlockSpec((C, W_DIM), lambda nn, ii: (0, 0)),
            pl.BlockSpec((1, XW), lambda nn, ii: (0, 0)),
            pl.BlockSpec((1, XW), lambda nn, ii: (0, 0)),
        ],
        out_specs=pl.BlockSpec((1, C, ROWS, W_DIM),
                               lambda nn, ii: (nn, 0, jnp.maximum(ii - 1, 0), 0)),
        out_shape=jax.ShapeDtypeStruct((n, C, H, W_DIM), jnp.float32),
        scratch_shapes=[pltpu.VMEM((2, 96, XW), jnp.float32)],
        compiler_params=pltpu.CompilerParams(
            dimension_semantics=("parallel", "arbitrary"),
            vmem_limit_bytes=100 * 1024 * 1024,
        ),
    )(x, w2, bb, mask_m, mask_p)


# speedup vs baseline: 1.0371x; 1.0371x over previous
import jax
import jax.numpy as jnp
from jax.experimental import pallas as pl
from jax.experimental.pallas import tpu as pltpu

ROWS = 64
H = 512
W_DIM = 512
C = 32
NBLK = H // ROWS
MARGIN = 128
NLANES = ROWS * W_DIM
NDATA = (ROWS + 2) * W_DIM
XW = MARGIN + NDATA + 2 * MARGIN


def _row(r):
    return MARGIN + r * W_DIM


def _conv_body(x_ref, w2_ref, b_ref, mm_ref, mp_ref, out_ref, slabs_ref):
    i = pl.program_id(1)

    @pl.when(i == 0)
    def _():
        slabs_ref[...] = jnp.zeros_like(slabs_ref)

    par = jax.lax.rem(i, 2)
    slab_ref = slabs_ref.at[1 - par]
    stage_ref = slabs_ref.at[par]

    xt = jnp.swapaxes(x_ref[0], 0, 1)

    @pl.when(i > 0)
    def _():
        @pl.when(i < NBLK)
        def _():
            slab_ref[C:2 * C, _row(ROWS + 1):_row(ROWS + 2)] = xt[0]

        @pl.when(i == NBLK)
        def _():
            slab_ref[C:2 * C, _row(ROWS + 1):_row(ROWS + 2)] = jnp.zeros(
                (C, W_DIM), jnp.float32)

        b1 = slab_ref[C:2 * C, :]
        slab_ref[0:C, :] = pltpu.roll(b1, 1, 1) * mm_ref[...]
        slab_ref[2 * C:3 * C, :] = pltpu.roll(b1, XW - 1, 1) * mp_ref[...]

        p = jax.lax.dot_general(
            w2_ref[...], slab_ref[:, MARGIN:MARGIN + NDATA],
            (((1,), (0,)), ((), ())),
            preferred_element_type=jnp.float32)
        bias = b_ref[...]
        out = (p[0:C, 0:NLANES]
               + p[C:2 * C, W_DIM:NLANES + W_DIM]
               + p[2 * C:3 * C, 2 * W_DIM:NLANES + 2 * W_DIM])
        res3 = jnp.stack(
            [out[:, r * W_DIM:(r + 1) * W_DIM] + bias for r in range(ROWS)],
            axis=0)
        out_ref[0] = jnp.swapaxes(res3, 0, 1)

    @pl.when(i < NBLK)
    def _():
        @pl.when(i > 0)
        def _():
            stage_ref[C:2 * C, _row(0):_row(1)] = (
                slab_ref[C:2 * C, _row(ROWS):_row(ROWS + 1)])

        for r in range(ROWS):
            stage_ref[C:2 * C, _row(r + 1):_row(r + 2)] = xt[r]


def kernel(x, W, b):
    n = x.shape[0]
    w2 = jnp.transpose(W, (2, 0, 3, 1)).reshape(96, 96).astype(jnp.float32)
    bb = jnp.broadcast_to(b[:, None], (C, W_DIM))
    lane = jnp.arange(XW, dtype=jnp.int32)[None, :]
    mask_m = jnp.where((lane - MARGIN) % W_DIM == 0, 0.0, 1.0).astype(jnp.float32)
    mask_p = jnp.where((lane - MARGIN) % W_DIM == W_DIM - 1, 0.0, 1.0).astype(
        jnp.float32)

    return pl.pallas_call(
        _conv_body,
        grid=(n, NBLK + 1),
        in_specs=[
            pl.BlockSpec((1, C, ROWS, W_DIM),
                         lambda nn, ii: (nn, 0, jnp.minimum(ii, NBLK - 1), 0)),
            pl.BlockSpec((96, 96), lambda nn, ii: (0, 0)),
            pl.BlockSpec((C, W_DIM), lambda nn, ii: (0, 0)),
            pl.BlockSpec((1, XW), lambda nn, ii: (0, 0)),
            pl.BlockSpec((1, XW), lambda nn, ii: (0, 0)),
        ],
        out_specs=pl.BlockSpec((1, C, ROWS, W_DIM),
                               lambda nn, ii: (nn, 0, jnp.maximum(ii - 1, 0), 0)),
        out_shape=jax.ShapeDtypeStruct((n, C, H, W_DIM), jnp.float32),
        scratch_shapes=[pltpu.VMEM((2, 96, XW), jnp.float32)],
        compiler_params=pltpu.CompilerParams(
            dimension_semantics=("parallel", "arbitrary"),
            vmem_limit_bytes=100 * 1024 * 1024,
        ),
    )(x, w2, bb, mask_m, mask_p)


# speedup vs baseline: 1.0622x; 1.0241x over previous
import jax
import jax.numpy as jnp
from jax.experimental import pallas as pl
from jax.experimental.pallas import tpu as pltpu

ROWS = 64
H = 512
W_DIM = 512
C = 32
NBLK = H // ROWS
MARGIN = 128
NLANES = ROWS * W_DIM
NDATA = (ROWS + 2) * W_DIM
XW = MARGIN + NDATA + 2 * MARGIN


def _row(r):
    return MARGIN + r * W_DIM


def _conv_body(x_ref, w2_ref, b_ref, mm_ref, mp_ref, out_ref, slab_ref):
    i = pl.program_id(1)

    @pl.when(i == 0)
    def _():
        slab_ref[...] = jnp.zeros_like(slab_ref)

    xt = jnp.swapaxes(x_ref[0], 0, 1)

    @pl.when(i > 0)
    def _():
        @pl.when(i < NBLK)
        def _():
            slab_ref[C:2 * C, _row(ROWS + 1):_row(ROWS + 2)] = xt[0]

        @pl.when(i == NBLK)
        def _():
            slab_ref[C:2 * C, _row(ROWS + 1):_row(ROWS + 2)] = jnp.zeros(
                (C, W_DIM), jnp.float32)

        b1 = slab_ref[C:2 * C, :]
        slab_ref[0:C, :] = pltpu.roll(b1, 1, 1) * mm_ref[...]
        slab_ref[2 * C:3 * C, :] = pltpu.roll(b1, XW - 1, 1) * mp_ref[...]

        p = jax.lax.dot_general(
            w2_ref[...], slab_ref[:, MARGIN:MARGIN + NDATA],
            (((1,), (0,)), ((), ())),
            preferred_element_type=jnp.float32)
        bias = b_ref[...]
        out = (p[0:C, 0:NLANES]
               + p[C:2 * C, W_DIM:NLANES + W_DIM]
               + p[2 * C:3 * C, 2 * W_DIM:NLANES + 2 * W_DIM])
        res3 = jnp.stack(
            [out[:, r * W_DIM:(r + 1) * W_DIM] + bias for r in range(ROWS)],
            axis=0)
        out_ref[0] = jnp.swapaxes(res3, 0, 1)

    @pl.when(i < NBLK)
    def _():
        @pl.when(i > 0)
        def _():
            slab_ref[C:2 * C, _row(0):_row(1)] = (
                slab_ref[C:2 * C, _row(ROWS):_row(ROWS + 1)])

        for r in range(ROWS):
            slab_ref[C:2 * C, _row(r + 1):_row(r + 2)] = xt[r]


def kernel(x, W, b):
    n = x.shape[0]
    w2 = jnp.transpose(W, (2, 0, 3, 1)).reshape(96, 96).astype(jnp.float32)
    bb = jnp.broadcast_to(b[:, None], (C, W_DIM))
    lane = jnp.arange(XW, dtype=jnp.int32)[None, :]
    mask_m = jnp.where((lane - MARGIN) % W_DIM == 0, 0.0, 1.0).astype(jnp.float32)
    mask_p = jnp.where((lane - MARGIN) % W_DIM == W_DIM - 1, 0.0, 1.0).astype(
        jnp.float32)

    return pl.pallas_call(
        _conv_body,
        grid=(n, NBLK + 1),
        in_specs=[
            pl.BlockSpec((1, C, ROWS, W_DIM),
                         lambda nn, ii: (nn, 0, jnp.minimum(ii, NBLK - 1), 0)),
            pl.BlockSpec((96, 96), lambda nn, ii: (0, 0)),
            pl.BlockSpec((C, W_DIM), lambda nn, ii: (0, 0)),
            pl.BlockSpec((1, XW), lambda nn, ii: (0, 0)),
            pl.BlockSpec((1, XW), lambda nn, ii: (0, 0)),
        ],
        out_specs=pl.BlockSpec((1, C, ROWS, W_DIM),
                               lambda nn, ii: (nn, 0, jnp.maximum(ii - 1, 0), 0)),
        out_shape=jax.ShapeDtypeStruct((n, C, H, W_DIM), jnp.float32),
        scratch_shapes=[pltpu.VMEM((96, XW), jnp.float32)],
        compiler_params=pltpu.CompilerParams(
            dimension_semantics=("parallel", "arbitrary"),
            vmem_limit_bytes=100 * 1024 * 1024,
        ),
    )(x, w2, bb, mask_m, mask_p)
